# baseline (speedup 1.0000x reference)
import functools

import jax
import jax.numpy as jnp
from jax import lax
from jax.experimental import pallas as pl
from jax.experimental.pallas import tpu as pltpu
from jax.experimental.pallas import tpu_sc as plsc

D_MODEL = 64
NUM_CORES = 2
NUM_SUBCORES = 16
NUM_WORKERS = NUM_CORES * NUM_SUBCORES
CHUNK = 128
NBUF = 4


def _emb_call(B, b_per_w, n_chunks):
    mesh = plsc.VectorSubcoreMesh(core_axis_name="c", subcore_axis_name="s")

    @functools.partial(
        pl.kernel,
        mesh=mesh,
        compiler_params=pltpu.CompilerParams(use_tc_tiling_on_sc=False),
        out_type=jax.ShapeDtypeStruct((B, D_MODEL), jnp.float32),
        scratch_types=[
            pltpu.VMEM((n_chunks, CHUNK), jnp.int32),
            pltpu.VMEM((NBUF, CHUNK, D_MODEL), jnp.float32),
            pltpu.SemaphoreType.DMA((NBUF,)),
        ],
    )
    def emb(table_hbm, idx_hbm, out_hbm, idx_v, rows_v, gsem):
        wid = lax.axis_index("s") * NUM_CORES + lax.axis_index("c")
        base = wid * b_per_w
        pltpu.sync_copy(idx_hbm.at[wid], idx_v)

        def gather(g, b):
            return pltpu.make_async_copy(
                table_hbm.at[idx_v.at[g]], rows_v.at[b], gsem.at[b])

        for b in range(NBUF):
            gather(b, b).start()

        def body(i, carry):
            for b in range(NBUF):
                g = i * NBUF + b
                gather(g, b).wait()
                pltpu.sync_copy(
                    rows_v.at[b], out_hbm.at[pl.ds(base + g * CHUNK, CHUNK)])
                gather(g + NBUF, b).start()
            return carry

        n_outer = n_chunks // NBUF
        lax.fori_loop(0, n_outer - 1, body, 0)
        for b in range(NBUF):
            g = (n_outer - 1) * NBUF + b
            gather(g, b).wait()
            pltpu.sync_copy(
                rows_v.at[b], out_hbm.at[pl.ds(base + g * CHUNK, CHUNK)])

    return emb


def kernel(x, W):
    n_rows, seq = x.shape
    B = n_rows * seq
    b_per_w = B // NUM_WORKERS
    n_chunks = b_per_w // CHUNK
    idx = x.reshape(NUM_WORKERS, n_chunks, CHUNK).astype(jnp.int32)
    out = _emb_call(B, b_per_w, n_chunks)(W, idx)
    return out.reshape(n_rows, seq, D_MODEL)
